# baseline (speedup 1.0000x reference)
import math

import jax
import jax.numpy as jnp
from jax import lax
from jax.experimental import pallas as pl
from jax.experimental.pallas import tpu as pltpu
from jax.experimental.pallas import tpu_sc as plsc

VOCAB = 100000
D_MODEL = 1024
SCALE = math.sqrt(D_MODEL)

NC = 2
NS = 16
NW = NC * NS
LANES = 16

B_TOTAL = 4 * 4096
B_PER_W = B_TOTAL // NW
CH = 40
N_FULL = 12
CH_TAIL = B_PER_W - N_FULL * CH
N_CH = N_FULL + 1
NBUF = 3


W_PER_G = 4096 // B_PER_W


def _emb_kernel(idx_hbm, table_hbm, out_hbm, idx_v, rows0, rows1, rows2,
                gsem0, gsem1, gsem2, ssem0, ssem1, ssem2):
    wid = lax.axis_index("s") * NC + lax.axis_index("c")
    g = wid // W_PER_G
    base = (wid % W_PER_G) * B_PER_W
    rows = (rows0, rows1, rows2)
    gsem = (gsem0, gsem1, gsem2)
    ssem = (ssem0, ssem1, ssem2)

    pltpu.sync_copy(idx_hbm.at[g, pl.ds(base, B_PER_W)], idx_v)

    def buf(b, size):
        return rows[b] if size == CH else rows[b].at[pl.ds(0, size)]

    def gather_desc(b, ci, size=CH):
        return pltpu.make_async_copy(
            table_hbm.at[idx_v.at[pl.ds(ci * CH, size)]], buf(b, size),
            gsem[b],
        )

    def scatter_desc(b, ci, size=CH):
        return pltpu.make_async_copy(
            buf(b, size), out_hbm.at[g, pl.ds(base + ci * CH, size)],
            ssem[b],
        )

    def scale_buf(b, size):
        def scale_row(r, _):
            for j in range(D_MODEL // LANES):
                col = j * LANES
                rows[b][r, pl.ds(col, LANES)] = (
                    rows[b][r, pl.ds(col, LANES)] * SCALE
                )
            return 0

        lax.fori_loop(0, size, scale_row, 0)

    gather_desc(0, 0).start()
    gather_desc(1, 1).start()

    def step(ci_base, k, refill=True, size=CH):
        ci = ci_base + k
        b = k
        gather_desc(b, ci, size).wait()
        scale_buf(b, size)
        scatter_desc(b, ci, size).start()

        if not refill:
            return
        nb = (k + 2) % NBUF

        @pl.when(ci == 0)
        def _():
            gather_desc(nb, ci + 2).start()

        @pl.when(jnp.logical_and(ci >= 1, ci + 2 < N_FULL))
        def _():
            scatter_desc(nb, ci - 1).wait()
            gather_desc(nb, ci + 2).start()

        @pl.when(ci + 2 == N_FULL)
        def _():
            scatter_desc(nb, ci - 1).wait()
            gather_desc(nb, ci + 2, CH_TAIL).start()

    def round_body(r, _):
        for k in range(NBUF):
            step(r * NBUF, k)
        return 0

    lax.fori_loop(0, N_FULL // NBUF, round_body, 0)
    step(N_FULL, 0, refill=False, size=CH_TAIL)

    scatter_desc(1, N_CH - 3).wait()
    scatter_desc(2, N_CH - 2).wait()
    scatter_desc(0, N_CH - 1, CH_TAIL).wait()


@jax.jit
def kernel(input, table):
    idx = input.astype(jnp.int32)
    mesh = plsc.VectorSubcoreMesh(core_axis_name="c", subcore_axis_name="s")
    return pl.kernel(
        _emb_kernel,
        out_type=jax.ShapeDtypeStruct(input.shape + (D_MODEL,), jnp.float32),
        mesh=mesh,
        scratch_types=(
            [pltpu.VMEM((B_PER_W,), jnp.int32)]
            + [pltpu.VMEM((CH, D_MODEL), jnp.float32)] * NBUF
            + [pltpu.SemaphoreType.DMA] * (2 * NBUF)
        ),
    )(idx, table)


# speedup vs baseline: 1.0088x; 1.0088x over previous
import math

import jax
import jax.numpy as jnp
from jax import lax
from jax.experimental import pallas as pl
from jax.experimental.pallas import tpu as pltpu
from jax.experimental.pallas import tpu_sc as plsc

VOCAB = 100000
D_MODEL = 1024
SCALE = math.sqrt(D_MODEL)

NC = 2
NS = 16
NW = NC * NS
LANES = 16

B_TOTAL = 4 * 4096
B_PER_W = B_TOTAL // NW
CH = 32
N_FULL = 15
CH_TAIL = B_PER_W - N_FULL * CH
N_CH = N_FULL + 1
NBUF = 3


W_PER_G = 4096 // B_PER_W


def _emb_kernel(idx_hbm, table_hbm, out_hbm, idx_v, rows0, rows1, rows2,
                gsem0, gsem1, gsem2, ssem0, ssem1, ssem2):
    wid = lax.axis_index("s") * NC + lax.axis_index("c")
    g = wid // W_PER_G
    base = (wid % W_PER_G) * B_PER_W
    rows = (rows0, rows1, rows2)
    gsem = (gsem0, gsem1, gsem2)
    ssem = (ssem0, ssem1, ssem2)

    pltpu.sync_copy(idx_hbm.at[g, pl.ds(base, B_PER_W)], idx_v)

    def buf(b, size):
        return rows[b] if size == CH else rows[b].at[pl.ds(0, size)]

    def gather_desc(b, ci, size=CH):
        return pltpu.make_async_copy(
            table_hbm.at[idx_v.at[pl.ds(ci * CH, size)]], buf(b, size),
            gsem[b],
        )

    def scatter_desc(b, ci, size=CH):
        return pltpu.make_async_copy(
            buf(b, size), out_hbm.at[g, pl.ds(base + ci * CH, size)],
            ssem[b],
        )

    def scale_buf(b, size):
        def scale_row(r, _):
            for j in range(D_MODEL // LANES):
                col = j * LANES
                rows[b][r, pl.ds(col, LANES)] = (
                    rows[b][r, pl.ds(col, LANES)] * SCALE
                )
            return 0

        lax.fori_loop(0, size, scale_row, 0)

    gather_desc(0, 0).start()
    gather_desc(1, 1).start()

    def step(ci_base, k, refill=True, size=CH):
        ci = ci_base + k
        b = k
        gather_desc(b, ci, size).wait()
        scale_buf(b, size)
        scatter_desc(b, ci, size).start()

        if not refill:
            return
        nb = (k + 2) % NBUF

        @pl.when(ci == 0)
        def _():
            gather_desc(nb, ci + 2).start()

        @pl.when(jnp.logical_and(ci >= 1, ci + 2 < N_FULL))
        def _():
            scatter_desc(nb, ci - 1).wait()
            gather_desc(nb, ci + 2).start()

        @pl.when(ci + 2 == N_FULL)
        def _():
            scatter_desc(nb, ci - 1).wait()
            gather_desc(nb, ci + 2, CH_TAIL).start()

    def round_body(r, _):
        for k in range(NBUF):
            step(r * NBUF, k)
        return 0

    lax.fori_loop(0, N_FULL // NBUF, round_body, 0)
    step(N_FULL, 0, refill=False, size=CH_TAIL)

    scatter_desc(1, N_CH - 3).wait()
    scatter_desc(2, N_CH - 2).wait()
    scatter_desc(0, N_CH - 1, CH_TAIL).wait()


@jax.jit
def kernel(input, table):
    idx = input.astype(jnp.int32)
    mesh = plsc.VectorSubcoreMesh(core_axis_name="c", subcore_axis_name="s")
    return pl.kernel(
        _emb_kernel,
        out_type=jax.ShapeDtypeStruct(input.shape + (D_MODEL,), jnp.float32),
        mesh=mesh,
        scratch_types=(
            [pltpu.VMEM((B_PER_W,), jnp.int32)]
            + [pltpu.VMEM((CH, D_MODEL), jnp.float32)] * NBUF
            + [pltpu.SemaphoreType.DMA] * (2 * NBUF)
        ),
    )(idx, table)


# speedup vs baseline: 1.0106x; 1.0018x over previous
import math

import jax
import jax.numpy as jnp
from jax import lax
from jax.experimental import pallas as pl
from jax.experimental.pallas import tpu as pltpu
from jax.experimental.pallas import tpu_sc as plsc

VOCAB = 100000
D_MODEL = 1024
SCALE = math.sqrt(D_MODEL)

NC = 2
NS = 16
NW = NC * NS
LANES = 16

B_TOTAL = 4 * 4096
B_PER_W = B_TOTAL // NW
CH = 32
N_FULL = 15
CH_TAIL = B_PER_W - N_FULL * CH
N_CH = N_FULL + 1
NBUF = 3


W_PER_G = 4096 // B_PER_W


def _emb_kernel(idx_hbm, table_hbm, out_hbm, idx_v, rows0, rows1, rows2,
                gsem0, gsem1, gsem2, ssem0, ssem1, ssem2):
    wid = lax.axis_index("s") * NC + lax.axis_index("c")
    g = wid // W_PER_G
    base = (wid % W_PER_G) * B_PER_W
    rows = (rows0, rows1, rows2)
    gsem = (gsem0, gsem1, gsem2)
    ssem = (ssem0, ssem1, ssem2)

    pltpu.sync_copy(idx_hbm.at[g, pl.ds(base, B_PER_W)], idx_v)

    def buf(b, size):
        return rows[b] if size == CH else rows[b].at[pl.ds(0, size)]

    def gather_desc(b, ci, size=CH):
        return pltpu.make_async_copy(
            table_hbm.at[idx_v.at[pl.ds(ci * CH, size)]], buf(b, size),
            gsem[b],
        )

    def scatter_desc(b, ci, size=CH):
        return pltpu.make_async_copy(
            buf(b, size), out_hbm.at[g, pl.ds(base + ci * CH, size)],
            ssem[b],
        )

    def scale_buf(b, size):
        def scale_row(r, _):
            for j in range(D_MODEL // LANES):
                col = j * LANES
                rows[b][r, pl.ds(col, LANES)] = (
                    rows[b][r, pl.ds(col, LANES)] * SCALE
                )
            return 0

        lax.fori_loop(0, size, scale_row, 0)

    gather_desc(0, 0).start()
    gather_desc(1, 1).start()

    def step(ci_base, k, refill=True, size=CH):
        ci = ci_base + k
        b = k
        gather_desc(b, ci, size).wait()
        scale_buf(b, size)
        scatter_desc(b, ci, size).start()

        if not refill:
            return
        nb = (k + 2) % NBUF

        @pl.when(ci == 0)
        def _():
            gather_desc(nb, ci + 2).start()

        @pl.when(jnp.logical_and(ci >= 1, ci + 2 < N_FULL))
        def _():
            scatter_desc(nb, ci - 1).wait()
            gather_desc(nb, ci + 2).start()

        @pl.when(ci + 2 == N_FULL)
        def _():
            scatter_desc(nb, ci - 1).wait()
            gather_desc(nb, ci + 2, CH_TAIL).start()

    def round_body(r, _):
        for k in range(NBUF):
            step(r * NBUF, k)
        return 0

    lax.fori_loop(0, N_FULL // NBUF, round_body, 0)
    step(N_FULL, 0, refill=False, size=CH_TAIL)

    scatter_desc(1, N_CH - 3).wait()
    scatter_desc(2, N_CH - 2).wait()
    scatter_desc(0, N_CH - 1, CH_TAIL).wait()


@jax.jit
def kernel(input, table):
    idx = input if input.dtype == jnp.int32 else input.astype(jnp.int32)
    mesh = plsc.VectorSubcoreMesh(core_axis_name="c", subcore_axis_name="s")
    return pl.kernel(
        _emb_kernel,
        out_type=jax.ShapeDtypeStruct(input.shape + (D_MODEL,), jnp.float32),
        mesh=mesh,
        scratch_types=(
            [pltpu.VMEM((B_PER_W,), jnp.int32)]
            + [pltpu.VMEM((CH, D_MODEL), jnp.float32)] * NBUF
            + [pltpu.SemaphoreType.DMA] * (2 * NBUF)
        ),
    )(idx, table)
